# Initial kernel scaffold; baseline (speedup 1.0000x reference)
import jax
import jax.numpy as jnp
from jax.experimental import pallas as pl


def kernel(u, s):
    raise NotImplementedError("write your pallas kernel here")


# speedup vs baseline: 1.5951x; 1.5951x over previous
import jax
import jax.numpy as jnp
from jax.experimental import pallas as pl
from jax.experimental.pallas import tpu as pltpu


def _c2q_body(u_ref, s_ref, o_ref):
    s = s_ref[...]
    m = jnp.max(s, axis=1, keepdims=True)
    e = jnp.exp(s - m)
    denom = jnp.sum(e, axis=1, keepdims=True)
    a = (e / denom).astype(jnp.bfloat16)
    o_ref[...] = jax.lax.dot_general(
        u_ref[...], a,
        dimension_numbers=(((0,), (1,)), ((), ())),
        preferred_element_type=jnp.float32,
    )


def kernel(u, s):
    t, j = s.shape
    d = u.shape[2]
    ub = u[0].astype(jnp.bfloat16)
    bt = 512
    return pl.pallas_call(
        _c2q_body,
        grid=(t // bt,),
        in_specs=[
            pl.BlockSpec((j, d), lambda i: (0, 0)),
            pl.BlockSpec((bt, j), lambda i: (i, 0)),
        ],
        out_specs=pl.BlockSpec((d, bt), lambda i: (0, i)),
        out_shape=jax.ShapeDtypeStruct((d, t), jnp.float32),
        compiler_params=pltpu.CompilerParams(
            dimension_semantics=("arbitrary",),
            vmem_limit_bytes=50 * 1024 * 1024,
        ),
        name="context2query_fused",
    )(ub, s)
